# baseline (speedup 1.0000x reference)
import jax
import jax.numpy as jnp
from jax import lax
from jax.experimental import pallas as pl
from jax.experimental.pallas import tpu as pltpu
from jax.experimental.pallas import tpu_sc as plsc

D = 768
D2 = D // 2
GK = D // 32
N_POS = 4096
B, S = 4, 4096
N = B * S
NC, NS = 2, 16
NW = NC * NS
PER_W = N // NW
WPB = S // PER_W
RPW = N_POS // NS
C = 16
NCH = PER_W // C
NBUF = 4
LANES = 16
SCALE = 4096.0
INV_SCALE = 1.0 / SCALE


def _body(inp_ref, pos_ref, tok_tab, pos_q, out_ref,
          idx_t, idx_p,
          tok0, tok1, tok2, tok3, pp0, pp1, pp2, pp3,
          st0, st1, st2, st3, sp0, sp1, sp2, sp3,
          sw0, sw1, sw2, sw3):
    cid = lax.axis_index("c")
    sid = lax.axis_index("s")
    wid = sid * NC + cid
    brow = wid // WPB
    col0 = (wid % WPB) * PER_W
    pltpu.sync_copy(inp_ref.at[brow, pl.ds(col0, PER_W)], idx_t)
    pltpu.sync_copy(pos_ref.at[brow, pl.ds(col0, PER_W)], idx_p)

    toks = (tok0, tok1, tok2, tok3)
    pps = (pp0, pp1, pp2, pp3)
    sts = (st0, st1, st2, st3)
    sps = (sp0, sp1, sp2, sp3)
    sws = (sw0, sw1, sw2, sw3)

    def gt_desc(j, b):
        return pltpu.make_async_copy(
            tok_tab.at[idx_t.at[pl.ds(j * C, C)]], toks[b], sts[b])

    def gp_desc(j, b):
        return pltpu.make_async_copy(
            pos_q.at[idx_p.at[pl.ds(j * C, C)]], pps[b], sps[b])

    def w_desc(j, b):
        return pltpu.make_async_copy(
            toks[b], out_ref.at[brow, pl.ds(col0 + j * C, C)], sws[b])

    gt_desc(0, 0).start()
    gp_desc(0, 0).start()
    gt_desc(1, 1).start()
    gp_desc(1, 1).start()

    def add(b):
        tb, pb = toks[b], pps[b]

        def add_row(r, _):
            for k in range(D2 // LANES):
                w = pb[r, pl.ds(k * LANES, LANES)]
                ev = ((w << 16) >> 16).astype(jnp.float32) * INV_SCALE
                od = (w >> 16).astype(jnp.float32) * INV_SCALE
                s0 = pl.ds(k * LANES, LANES)
                s1 = pl.ds(D2 + k * LANES, LANES)
                tb[r, s0] = tb[r, s0] + ev
                tb[r, s1] = tb[r, s1] + od
            return 0

        lax.fori_loop(0, C, add_row, 0)

    def step(j, b, issue_ahead=True, wait_wb=True):
        gt_desc(j, b).wait()
        gp_desc(j, b).wait()
        if wait_wb:
            w_desc(j - 2, (b - 2) % NBUF).wait()
        if issue_ahead:
            nb = (b + 2) % NBUF
            gt_desc(j + 2, nb).start()
            gp_desc(j + 2, nb).start()
        add(b)
        w_desc(j, b).start()

    step(0, 0, wait_wb=False)
    step(1, 1, wait_wb=False)

    def mid(j2, _):
        jbase = 2 + j2 * NBUF
        for i in range(NBUF):
            step(jbase + i, (2 + i) % NBUF)
        return 0

    lax.fori_loop(0, 7, mid, 0)

    step(30, 30 % NBUF, issue_ahead=False)
    step(31, 31 % NBUF, issue_ahead=False)
    w_desc(NCH - 2, (NCH - 2) % NBUF).wait()
    w_desc(NCH - 1, (NCH - 1) % NBUF).wait()


@jax.jit
def kernel(input, pos, token_table, pos_table):
    q = jnp.clip(jnp.round(pos_table * SCALE), -32768.0, 32767.0)
    q16 = q.astype(jnp.int16).reshape(N_POS, 2, D2).swapaxes(1, 2)
    pos_q = jax.lax.bitcast_convert_type(q16, jnp.int32)
    mesh = plsc.VectorSubcoreMesh(core_axis_name="c", subcore_axis_name="s")
    k = pl.kernel(
        _body,
        mesh=mesh,
        out_type=jax.ShapeDtypeStruct((B, S, D), jnp.float32),
        scratch_types=(
            [pltpu.VMEM((PER_W,), jnp.int32)] * 2
            + [pltpu.VMEM((C, D), jnp.float32)] * NBUF
            + [pltpu.VMEM((C, D2), jnp.int32)] * NBUF
            + [pltpu.SemaphoreType.DMA] * (3 * NBUF)
        ),
    )
    return k(input, pos, token_table, pos_q)


# speedup vs baseline: 3.0467x; 3.0467x over previous
import jax
import jax.numpy as jnp
from jax import lax
from jax.experimental import pallas as pl
from jax.experimental.pallas import tpu as pltpu
from jax.experimental.pallas import tpu_sc as plsc

D = 768
B, S = 4, 4096
N = B * S
NC, NS = 2, 16
NW = NC * NS
PER_W = N // NW
WPB = S // PER_W
C = 16
NCH = PER_W // C
NBUF = 4
LANES = 16
COLS = D // LANES


def _body(inp_ref, pos_ref, tok_tab, pos_tab, out_ref,
          idx_t, idx_p,
          tok0, tok1, tok2, tok3, pb0, pb1, pb2, pb3,
          st0, st1, st2, st3, sp0, sp1, sp2, sp3,
          sw0, sw1, sw2, sw3):
    wid = lax.axis_index("s") * NC + lax.axis_index("c")
    brow = wid // WPB
    col0 = (wid % WPB) * PER_W
    pltpu.sync_copy(inp_ref.at[brow, pl.ds(col0, PER_W)], idx_t)
    pltpu.sync_copy(pos_ref.at[brow, pl.ds(col0, PER_W)], idx_p)

    toks = (tok0, tok1, tok2, tok3)
    pbufs = (pb0, pb1, pb2, pb3)
    sts = (st0, st1, st2, st3)
    sps = (sp0, sp1, sp2, sp3)
    sws = (sw0, sw1, sw2, sw3)

    def g_descs(j, b):
        ct = pltpu.make_async_copy(
            tok_tab.at[idx_t.at[pl.ds(j * C, C)]], toks[b], sts[b])
        cp = pltpu.make_async_copy(
            pos_tab.at[idx_p.at[pl.ds(j * C, C)]], pbufs[b], sps[b])
        return ct, cp

    def g_issue(j, b):
        ct, cp = g_descs(j, b)
        ct.start()
        cp.start()

    def g_wait(j, b):
        ct, cp = g_descs(j, b)
        ct.wait()
        cp.wait()

    def w_desc(j, b):
        return pltpu.make_async_copy(
            toks[b], out_ref.at[brow, pl.ds(col0 + j * C, C)], sws[b])

    def add(b):
        tb, pb = toks[b], pbufs[b]

        def add_row(r, _):
            for k in range(COLS):
                s = pl.ds(k * LANES, LANES)
                tb[r, s] = tb[r, s] + pb[r, s]
            return 0

        lax.fori_loop(0, C, add_row, 0)

    def step(j, b, issue_ahead=True, wait_wb=True):
        g_wait(j, b)
        if wait_wb:
            w_desc(j - 2, (b - 2) % NBUF).wait()
        if issue_ahead:
            g_issue(j + 2, (b + 2) % NBUF)
        add(b)
        w_desc(j, b).start()

    g_issue(0, 0)
    g_issue(1, 1)
    step(0, 0, wait_wb=False)
    step(1, 1, wait_wb=False)

    def mid(j2, _):
        jbase = 2 + j2 * NBUF
        for i in range(NBUF):
            step(jbase + i, (2 + i) % NBUF)
        return 0

    lax.fori_loop(0, 7, mid, 0)

    step(30, 30 % NBUF, issue_ahead=False)
    step(31, 31 % NBUF, issue_ahead=False)
    w_desc(NCH - 2, (NCH - 2) % NBUF).wait()
    w_desc(NCH - 1, (NCH - 1) % NBUF).wait()


@jax.jit
def kernel(input, pos, token_table, pos_table):
    mesh = plsc.VectorSubcoreMesh(core_axis_name="c", subcore_axis_name="s")
    k = pl.kernel(
        _body,
        mesh=mesh,
        out_type=jax.ShapeDtypeStruct((B, S, D), jnp.float32),
        scratch_types=(
            [pltpu.VMEM((PER_W,), jnp.int32)] * 2
            + [pltpu.VMEM((C, D), jnp.float32)] * (2 * NBUF)
            + [pltpu.SemaphoreType.DMA] * (3 * NBUF)
        ),
    )
    return k(input, pos, token_table, pos_table)
